# baseline (speedup 1.0000x reference)
import functools

import jax
import jax.numpy as jnp
from jax import lax
from jax.experimental import pallas as pl
from jax.experimental.pallas import tpu as pltpu
from jax.experimental.pallas import tpu_sc as plsc

N = 10000
E = 320000
D = 128

NC = 2
NS = 16
L = 16

EW = E // (NC * NS)
C = 80
KCH = EW // C
RT = N // NS
ZB = 25


def _sc_aggregate(node_embs, edge_embs, src_idx, dst_idx):
    mesh = plsc.VectorSubcoreMesh(
        core_axis_name="c", subcore_axis_name="s",
        num_cores=NC, num_subcores=NS)

    out_type = [
        jax.ShapeDtypeStruct((NC, N, D), jnp.float32),
        jax.ShapeDtypeStruct((NC, N, D), jnp.float32),
        jax.ShapeDtypeStruct((NC, N, L), jnp.float32),
        jax.ShapeDtypeStruct((NC, N, L), jnp.float32),
    ]
    scratch = [
        pltpu.VMEM((2, C), jnp.int32),
        pltpu.VMEM((2, C), jnp.int32),
        pltpu.VMEM((2, C, D), jnp.float32),
        pltpu.VMEM((C, D), jnp.float32),
        pltpu.VMEM((C, L), jnp.float32),
        pltpu.VMEM((ZB, D), jnp.float32),
        pltpu.VMEM((ZB, L), jnp.float32),
        pltpu.VMEM_SHARED((N, D), jnp.float32),
        pltpu.VMEM_SHARED((N, L), jnp.float32),
        pltpu.SemaphoreType.DMA((2,)),
        pltpu.SemaphoreType.DMA((2,)),
        pltpu.SemaphoreType.DMA((2,)),
        pltpu.SemaphoreType.DMA((2,)),
        pltpu.SemaphoreType.DMA((2,)),
        pltpu.SemaphoreType.DMA,
    ]

    @functools.partial(
        pl.kernel, out_type=out_type, mesh=mesh, scratch_types=scratch,
        compiler_params=pltpu.CompilerParams(use_tc_tiling_on_sc=False))
    def body(node_hbm, edge_hbm, src_hbm, dst_hbm,
             out_o, out_i, out_do, out_di,
             gidx, sidx, gat, ech, ones, zbuf, zdbuf, acc, accd,
             gsem, igsem, issem, sasem, sdsem, esem):
        c = lax.axis_index("c")
        s = lax.axis_index("s")
        wid = c * NS + s
        ebase = wid * EW

        lanes = lax.iota(jnp.int32, L)
        zvec = jnp.zeros((L,), jnp.float32)
        onecol = jnp.where(lanes == 0, 1.0, 0.0)

        def initrow(r, _):
            ones[r, pl.ds(0, L)] = onecol
            return _
        lax.fori_loop(0, C, initrow, 0)

        def zrow(r, _):
            for cb in range(D // L):
                zbuf[r, pl.ds(cb * L, L)] = zvec
            zdbuf[r, pl.ds(0, L)] = zvec
            return _
        lax.fori_loop(0, ZB, zrow, 0)

        def run_phase(g_hbm, s_hbm, out_hbm, outd_hbm):
            for k in range(RT // ZB):
                pltpu.sync_copy(zbuf, acc.at[pl.ds(s * RT + k * ZB, ZB)])
                pltpu.sync_copy(zdbuf, accd.at[pl.ds(s * RT + k * ZB, ZB)])
            plsc.subcore_barrier()

            def issue_gidx(k, p):
                pltpu.async_copy(g_hbm.at[pl.ds(ebase + k * C, C)],
                                 gidx.at[p], igsem.at[p])

            def issue_sidx(k, p):
                pltpu.async_copy(s_hbm.at[pl.ds(ebase + k * C, C)],
                                 sidx.at[p], issem.at[p])

            def issue_gather(p):
                pltpu.async_copy(node_hbm.at[gidx.at[p]], gat.at[p],
                                 gsem.at[p])

            def issue_edge(k):
                pltpu.async_copy(edge_hbm.at[pl.ds(ebase + k * C, C)],
                                 ech, esem)

            def wait(src, dst, sem):
                pltpu.make_async_copy(src, dst, sem).wait()

            def subtract(p):
                @plsc.parallel_loop(0, C, 1, unroll=4)
                def _(r):
                    for cb in range(D // L):
                        sl = pl.ds(cb * L, L)
                        gat[p, r, sl] = gat[p, r, sl] - ech[r, sl]

            def issue_scatter(p):
                pltpu.async_copy(gat.at[p], acc.at[sidx.at[p]], sasem.at[p],
                                 add=True)
                pltpu.async_copy(ones, accd.at[sidx.at[p]], sdsem.at[p],
                                 add=True)

            def wait_scatter(p):
                wait(gat.at[p], acc.at[sidx.at[p]], sasem.at[p])
                wait(ones, accd.at[sidx.at[p]], sdsem.at[p])

            pltpu.sync_copy(g_hbm.at[pl.ds(ebase, C)], gidx.at[0])
            pltpu.sync_copy(s_hbm.at[pl.ds(ebase, C)], sidx.at[0])
            issue_gather(0)
            issue_edge(0)
            issue_gidx(1, 1)
            wait(node_hbm.at[gidx.at[0]], gat.at[0], gsem.at[0])
            wait(g_hbm.at[pl.ds(ebase, C)], gidx.at[1], igsem.at[1])
            issue_gather(1)
            issue_gidx(2, 0)
            issue_sidx(1, 1)
            wait(edge_hbm.at[pl.ds(ebase, C)], ech, esem)
            subtract(0)
            issue_edge(1)
            issue_scatter(0)

            def step(k, p, t, guard_g, guard_i):
                q = 1 - p
                wait(node_hbm.at[gidx.at[p]], gat.at[p], gsem.at[p])
                wait(edge_hbm.at[pl.ds(ebase, C)], ech, esem)
                subtract(p)

                def next_edge():
                    issue_edge(k + 1)
                if guard_g is None:
                    next_edge()
                else:
                    pl.when(guard_g)(next_edge)

                wait_scatter(q)

                def prefetch_g():
                    wait(g_hbm.at[pl.ds(ebase, C)], gidx.at[q], igsem.at[q])
                    issue_gather(q)
                    issue_sidx(k + 1, q)
                if guard_g is None:
                    prefetch_g()
                else:
                    pl.when(guard_g)(prefetch_g)

                def prefetch_i():
                    issue_gidx(k + 2, p)
                if guard_i is None:
                    prefetch_i()
                else:
                    pl.when(guard_i)(prefetch_i)

                wait(s_hbm.at[pl.ds(ebase, C)], sidx.at[p], issem.at[p])
                issue_scatter(p)

            def pair(t, _):
                g = t < (KCH - 3) // 2
                k1 = 2 * t + 1
                step(k1, 1, t, None, g)
                step(k1 + 1, 0, t, g, g)
                return _
            lax.fori_loop(0, (KCH - 1) // 2, pair, 0)

            wait_scatter(0)
            plsc.subcore_barrier()

            r0 = s * RT
            pltpu.sync_copy(acc.at[pl.ds(r0, RT)],
                            out_hbm.at[c, pl.ds(r0, RT)])
            pltpu.sync_copy(accd.at[pl.ds(r0, RT)],
                            outd_hbm.at[c, pl.ds(r0, RT)])
            plsc.subcore_barrier()

        run_phase(src_hbm, dst_hbm, out_o, out_do)
        run_phase(dst_hbm, src_hbm, out_i, out_di)

    return body(node_embs, edge_embs, src_idx, dst_idx)


def _tc_combine(acc_o, acc_i, deg_o, deg_i, node_embs,
                W_O_w, W_I_w, W_S_w, bias_sum, gamma, beta):
    def body(ao_ref, ai_ref, do_ref, di_ref, nd_ref, wo_ref, wi_ref, ws_ref,
             bs_ref, g_ref, b_ref, out_ref):
        deg_in = jnp.maximum(do_ref[0, :, 0:1] + do_ref[1, :, 0:1], 1.0)
        deg_out = jnp.maximum(di_ref[0, :, 0:1] + di_ref[1, :, 0:1], 1.0)
        ho = (ao_ref[0] + ao_ref[1]) / deg_in
        hi = (ai_ref[0] + ai_ref[1]) / deg_out
        h = (jnp.dot(ho, wo_ref[...].T, preferred_element_type=jnp.float32)
             + jnp.dot(hi, wi_ref[...].T, preferred_element_type=jnp.float32)
             + jnp.dot(nd_ref[...], ws_ref[...].T,
                       preferred_element_type=jnp.float32)
             + bs_ref[...]) / 3.0
        m = jnp.mean(h, axis=0, keepdims=True)
        hc = h - m
        v = jnp.mean(hc * hc, axis=0, keepdims=True)
        out_ref[...] = hc * lax.rsqrt(v + 1e-5) * g_ref[...] + b_ref[...]

    return pl.pallas_call(
        body,
        out_shape=jax.ShapeDtypeStruct((N, D), jnp.float32),
    )(acc_o, acc_i, deg_o, deg_i, node_embs, W_O_w, W_I_w, W_S_w,
      bias_sum, gamma, beta)


def kernel(node_embs, edge_embs, edge_index, W_O_w, W_O_b, W_I_w, W_I_b,
           W_S_w, W_S_b, gamma, beta):
    acc_o, acc_i, deg_o, deg_i = _sc_aggregate(
        node_embs, edge_embs, edge_index[0], edge_index[1])
    bias_sum = (W_O_b + W_I_b + W_S_b).reshape(1, D)
    return _tc_combine(acc_o, acc_i, deg_o, deg_i, node_embs,
                       W_O_w, W_I_w, W_S_w, bias_sum,
                       gamma.reshape(1, D), beta.reshape(1, D))


# speedup vs baseline: 1.1605x; 1.1605x over previous
import functools

import jax
import jax.numpy as jnp
from jax import lax
from jax.experimental import pallas as pl
from jax.experimental.pallas import tpu as pltpu
from jax.experimental.pallas import tpu_sc as plsc

N = 10000
E = 320000
D = 128

NC = 2
NS = 16
L = 16

EW = E // (NC * NS)
C = 80
KCH = EW // C
RT = N // NS
ZB = 25


def _sc_aggregate(node_embs, edge_embs, src_idx, dst_idx):
    mesh = plsc.VectorSubcoreMesh(
        core_axis_name="c", subcore_axis_name="s",
        num_cores=NC, num_subcores=NS)

    out_type = [
        jax.ShapeDtypeStruct((NC, N, D), jnp.float32),
        jax.ShapeDtypeStruct((NC, N, D), jnp.float32),
        jax.ShapeDtypeStruct((NC, N, L), jnp.float32),
        jax.ShapeDtypeStruct((NC, N, L), jnp.float32),
    ]
    scratch = [
        pltpu.VMEM((2, C), jnp.int32),
        pltpu.VMEM((2, C), jnp.int32),
        pltpu.VMEM((2, C, D), jnp.float32),
        pltpu.VMEM((C, D), jnp.float32),
        pltpu.VMEM((C, L), jnp.float32),
        pltpu.VMEM((ZB, D), jnp.float32),
        pltpu.VMEM((ZB, L), jnp.float32),
        pltpu.VMEM_SHARED((N, D), jnp.float32),
        pltpu.VMEM_SHARED((N, L), jnp.float32),
        pltpu.SemaphoreType.DMA((2,)),
        pltpu.SemaphoreType.DMA((2,)),
        pltpu.SemaphoreType.DMA((2,)),
        pltpu.SemaphoreType.DMA((2,)),
        pltpu.SemaphoreType.DMA((2,)),
        pltpu.SemaphoreType.DMA,
    ]

    @functools.partial(
        pl.kernel, out_type=out_type, mesh=mesh, scratch_types=scratch,
        compiler_params=pltpu.CompilerParams(use_tc_tiling_on_sc=False))
    def body(node_hbm, edge_hbm, src_hbm, dst_hbm,
             out_o, out_i, out_do, out_di,
             gidx, sidx, gat, ech, ones, zbuf, zdbuf, acc, accd,
             gsem, igsem, issem, sasem, sdsem, esem):
        c = lax.axis_index("c")
        s = lax.axis_index("s")
        wid = c * NS + s
        ebase = wid * EW

        lanes = lax.iota(jnp.int32, L)
        zvec = jnp.zeros((L,), jnp.float32)
        onecol = jnp.where(lanes == 0, 1.0, 0.0)

        def initrow(r, _):
            ones[r, pl.ds(0, L)] = onecol
            return _
        lax.fori_loop(0, C, initrow, 0)

        def zrow(r, _):
            for cb in range(D // L):
                zbuf[r, pl.ds(cb * L, L)] = zvec
            zdbuf[r, pl.ds(0, L)] = zvec
            return _
        lax.fori_loop(0, ZB, zrow, 0)

        def run_phase(g_hbm, s_hbm, out_hbm, outd_hbm):
            for k in range(RT // ZB):
                pltpu.sync_copy(zbuf, acc.at[pl.ds(s * RT + k * ZB, ZB)])
                pltpu.sync_copy(zdbuf, accd.at[pl.ds(s * RT + k * ZB, ZB)])
            plsc.subcore_barrier()

            def issue_gidx(k, p):
                pltpu.async_copy(g_hbm.at[pl.ds(ebase + k * C, C)],
                                 gidx.at[p], igsem.at[p])

            def issue_sidx(k, p):
                pltpu.async_copy(s_hbm.at[pl.ds(ebase + k * C, C)],
                                 sidx.at[p], issem.at[p])

            def issue_gather(p):
                pltpu.async_copy(node_hbm.at[gidx.at[p]], gat.at[p],
                                 gsem.at[p])

            def issue_edge(k):
                pltpu.async_copy(edge_hbm.at[pl.ds(ebase + k * C, C)],
                                 ech, esem)

            def wait(src, dst, sem):
                pltpu.make_async_copy(src, dst, sem).wait()

            def subtract(p):
                @plsc.parallel_loop(0, C, 1, unroll=8)
                def _(r):
                    for cb in range(D // L):
                        sl = pl.ds(cb * L, L)
                        gat[p, r, sl] = gat[p, r, sl] - ech[r, sl]

            def issue_scatter(p):
                pltpu.async_copy(gat.at[p], acc.at[sidx.at[p]], sasem.at[p],
                                 add=True)
                pltpu.async_copy(ones, accd.at[sidx.at[p]], sdsem.at[p],
                                 add=True)

            def wait_scatter(p):
                wait(gat.at[p], acc.at[sidx.at[p]], sasem.at[p])
                wait(ones, accd.at[sidx.at[p]], sdsem.at[p])

            pltpu.sync_copy(g_hbm.at[pl.ds(ebase, C)], gidx.at[0])
            pltpu.sync_copy(s_hbm.at[pl.ds(ebase, C)], sidx.at[0])
            issue_gather(0)
            issue_edge(0)
            issue_gidx(1, 1)
            wait(node_hbm.at[gidx.at[0]], gat.at[0], gsem.at[0])
            wait(g_hbm.at[pl.ds(ebase, C)], gidx.at[1], igsem.at[1])
            issue_gather(1)
            issue_gidx(2, 0)
            issue_sidx(1, 1)
            wait(edge_hbm.at[pl.ds(ebase, C)], ech, esem)
            subtract(0)
            issue_edge(1)
            issue_scatter(0)

            def step(k, p, t, guard_g, guard_i):
                q = 1 - p
                wait(node_hbm.at[gidx.at[p]], gat.at[p], gsem.at[p])
                wait_scatter(q)

                def prefetch_g():
                    wait(g_hbm.at[pl.ds(ebase, C)], gidx.at[q], igsem.at[q])
                    issue_gather(q)
                    issue_sidx(k + 1, q)
                if guard_g is None:
                    prefetch_g()
                else:
                    pl.when(guard_g)(prefetch_g)

                def prefetch_i():
                    issue_gidx(k + 2, p)
                if guard_i is None:
                    prefetch_i()
                else:
                    pl.when(guard_i)(prefetch_i)

                wait(edge_hbm.at[pl.ds(ebase, C)], ech, esem)
                subtract(p)

                def next_edge():
                    issue_edge(k + 1)
                if guard_g is None:
                    next_edge()
                else:
                    pl.when(guard_g)(next_edge)

                wait(s_hbm.at[pl.ds(ebase, C)], sidx.at[p], issem.at[p])
                issue_scatter(p)

            def pair(t, _):
                g = t < (KCH - 3) // 2
                k1 = 2 * t + 1
                step(k1, 1, t, None, g)
                step(k1 + 1, 0, t, g, g)
                return _
            lax.fori_loop(0, (KCH - 1) // 2, pair, 0)

            wait_scatter(0)
            plsc.subcore_barrier()

            r0 = s * RT
            pltpu.sync_copy(acc.at[pl.ds(r0, RT)],
                            out_hbm.at[c, pl.ds(r0, RT)])
            pltpu.sync_copy(accd.at[pl.ds(r0, RT)],
                            outd_hbm.at[c, pl.ds(r0, RT)])
            plsc.subcore_barrier()

        run_phase(src_hbm, dst_hbm, out_o, out_do)
        run_phase(dst_hbm, src_hbm, out_i, out_di)

    return body(node_embs, edge_embs, src_idx, dst_idx)


def _tc_combine(acc_o, acc_i, deg_o, deg_i, node_embs,
                W_O_w, W_I_w, W_S_w, bias_sum, gamma, beta):
    def body(ao_ref, ai_ref, do_ref, di_ref, nd_ref, wo_ref, wi_ref, ws_ref,
             bs_ref, g_ref, b_ref, out_ref):
        deg_in = jnp.maximum(do_ref[0, :, 0:1] + do_ref[1, :, 0:1], 1.0)
        deg_out = jnp.maximum(di_ref[0, :, 0:1] + di_ref[1, :, 0:1], 1.0)
        ho = (ao_ref[0] + ao_ref[1]) / deg_in
        hi = (ai_ref[0] + ai_ref[1]) / deg_out
        h = (jnp.dot(ho, wo_ref[...].T, preferred_element_type=jnp.float32)
             + jnp.dot(hi, wi_ref[...].T, preferred_element_type=jnp.float32)
             + jnp.dot(nd_ref[...], ws_ref[...].T,
                       preferred_element_type=jnp.float32)
             + bs_ref[...]) / 3.0
        m = jnp.mean(h, axis=0, keepdims=True)
        hc = h - m
        v = jnp.mean(hc * hc, axis=0, keepdims=True)
        out_ref[...] = hc * lax.rsqrt(v + 1e-5) * g_ref[...] + b_ref[...]

    return pl.pallas_call(
        body,
        out_shape=jax.ShapeDtypeStruct((N, D), jnp.float32),
    )(acc_o, acc_i, deg_o, deg_i, node_embs, W_O_w, W_I_w, W_S_w,
      bias_sum, gamma, beta)


def kernel(node_embs, edge_embs, edge_index, W_O_w, W_O_b, W_I_w, W_I_b,
           W_S_w, W_S_b, gamma, beta):
    acc_o, acc_i, deg_o, deg_i = _sc_aggregate(
        node_embs, edge_embs, edge_index[0], edge_index[1])
    bias_sum = (W_O_b + W_I_b + W_S_b).reshape(1, D)
    return _tc_combine(acc_o, acc_i, deg_o, deg_i, node_embs,
                       W_O_w, W_I_w, W_S_w, bias_sum,
                       gamma.reshape(1, D), beta.reshape(1, D))


# speedup vs baseline: 1.1946x; 1.0294x over previous
import functools

import jax
import jax.numpy as jnp
from jax import lax
from jax.experimental import pallas as pl
from jax.experimental.pallas import tpu as pltpu
from jax.experimental.pallas import tpu_sc as plsc

N = 10000
E = 320000
D = 128

NC = 2
NS = 16
L = 16

EW = E // NS
C = 80
KCH = EW // C
RT = N // NS
ZB = 25


def _sc_aggregate(node_embs, edge_embs, gcat, scat):
    mesh = plsc.VectorSubcoreMesh(
        core_axis_name="c", subcore_axis_name="s",
        num_cores=NC, num_subcores=NS)

    out_type = [
        jax.ShapeDtypeStruct((NC, N, D), jnp.float32),
        jax.ShapeDtypeStruct((NC, N, L), jnp.float32),
    ]
    scratch = [
        pltpu.VMEM((2, C), jnp.int32),
        pltpu.VMEM((2, C), jnp.int32),
        pltpu.VMEM((2, C, D), jnp.float32),
        pltpu.VMEM((C, D), jnp.float32),
        pltpu.VMEM((C, L), jnp.float32),
        pltpu.VMEM((ZB, D), jnp.float32),
        pltpu.VMEM((ZB, L), jnp.float32),
        pltpu.VMEM_SHARED((N, D), jnp.float32),
        pltpu.VMEM_SHARED((N, L), jnp.float32),
        pltpu.SemaphoreType.DMA((2,)),
        pltpu.SemaphoreType.DMA((2,)),
        pltpu.SemaphoreType.DMA((2,)),
        pltpu.SemaphoreType.DMA((2,)),
        pltpu.SemaphoreType.DMA((2,)),
        pltpu.SemaphoreType.DMA,
    ]

    @functools.partial(
        pl.kernel, out_type=out_type, mesh=mesh, scratch_types=scratch,
        compiler_params=pltpu.CompilerParams(use_tc_tiling_on_sc=False))
    def body(node_hbm, edge_hbm, g_hbm, s_hbm,
             out_hbm, outd_hbm,
             gidx, sidx, gat, ech, ones, zbuf, zdbuf, acc, accd,
             gsem, igsem, issem, sasem, sdsem, esem):
        c = lax.axis_index("c")
        s = lax.axis_index("s")
        ebase = c * E + s * EW
        eoff = s * EW

        lanes = lax.iota(jnp.int32, L)
        zvec = jnp.zeros((L,), jnp.float32)
        onecol = jnp.where(lanes == 0, 1.0, 0.0)

        def initrow(r, _):
            ones[r, pl.ds(0, L)] = onecol
            return _
        lax.fori_loop(0, C, initrow, 0)

        def zrow(r, _):
            for cb in range(D // L):
                zbuf[r, pl.ds(cb * L, L)] = zvec
            zdbuf[r, pl.ds(0, L)] = zvec
            return _
        lax.fori_loop(0, ZB, zrow, 0)

        if True:
            for k in range(RT // ZB):
                pltpu.sync_copy(zbuf, acc.at[pl.ds(s * RT + k * ZB, ZB)])
                pltpu.sync_copy(zdbuf, accd.at[pl.ds(s * RT + k * ZB, ZB)])
            plsc.subcore_barrier()

            def issue_gidx(k, p):
                pltpu.async_copy(g_hbm.at[pl.ds(ebase + k * C, C)],
                                 gidx.at[p], igsem.at[p])

            def issue_sidx(k, p):
                pltpu.async_copy(s_hbm.at[pl.ds(ebase + k * C, C)],
                                 sidx.at[p], issem.at[p])

            def issue_gather(p):
                pltpu.async_copy(node_hbm.at[gidx.at[p]], gat.at[p],
                                 gsem.at[p])

            def issue_edge(k):
                pltpu.async_copy(edge_hbm.at[pl.ds(eoff + k * C, C)],
                                 ech, esem)

            def wait(src, dst, sem):
                pltpu.make_async_copy(src, dst, sem).wait()

            def subtract(p):
                @plsc.parallel_loop(0, C, 1, unroll=4)
                def _(r):
                    for cb in range(D // L):
                        sl = pl.ds(cb * L, L)
                        gat[p, r, sl] = gat[p, r, sl] - ech[r, sl]

            def issue_scatter(p):
                pltpu.async_copy(gat.at[p], acc.at[sidx.at[p]], sasem.at[p],
                                 add=True)
                pltpu.async_copy(ones, accd.at[sidx.at[p]], sdsem.at[p],
                                 add=True)

            def wait_scatter(p):
                wait(gat.at[p], acc.at[sidx.at[p]], sasem.at[p])
                wait(ones, accd.at[sidx.at[p]], sdsem.at[p])

            pltpu.sync_copy(g_hbm.at[pl.ds(ebase, C)], gidx.at[0])
            pltpu.sync_copy(s_hbm.at[pl.ds(ebase, C)], sidx.at[0])
            issue_gather(0)
            issue_edge(0)
            issue_gidx(1, 1)
            wait(node_hbm.at[gidx.at[0]], gat.at[0], gsem.at[0])
            wait(g_hbm.at[pl.ds(ebase, C)], gidx.at[1], igsem.at[1])
            issue_gather(1)
            issue_gidx(2, 0)
            issue_sidx(1, 1)
            wait(edge_hbm.at[pl.ds(eoff, C)], ech, esem)
            subtract(0)
            issue_edge(1)
            issue_scatter(0)

            def step(k, p, guard_i):
                q = 1 - p
                wait(node_hbm.at[gidx.at[p]], gat.at[p], gsem.at[p])
                wait_scatter(q)

                wait(g_hbm.at[pl.ds(ebase, C)], gidx.at[q], igsem.at[q])
                issue_gather(q)
                issue_sidx(k + 1, q)

                def prefetch_i():
                    issue_gidx(k + 2, p)
                if guard_i is None:
                    prefetch_i()
                else:
                    pl.when(guard_i)(prefetch_i)

                wait(edge_hbm.at[pl.ds(eoff, C)], ech, esem)
                subtract(p)
                issue_edge(k + 1)

                wait(s_hbm.at[pl.ds(ebase, C)], sidx.at[p], issem.at[p])
                issue_scatter(p)

            def pair(t, _):
                k1 = 2 * t + 1
                step(k1, 1, None)
                step(k1 + 1, 0, t < (KCH - 4) // 2)
                return _
            lax.fori_loop(0, (KCH - 2) // 2, pair, 0)

            wait(node_hbm.at[gidx.at[1]], gat.at[1], gsem.at[1])
            wait_scatter(0)
            wait(edge_hbm.at[pl.ds(eoff, C)], ech, esem)
            subtract(1)
            wait(s_hbm.at[pl.ds(ebase, C)], sidx.at[1], issem.at[1])
            issue_scatter(1)
            wait_scatter(1)
            plsc.subcore_barrier()

            r0 = s * RT
            pltpu.sync_copy(acc.at[pl.ds(r0, RT)],
                            out_hbm.at[c, pl.ds(r0, RT)])
            pltpu.sync_copy(accd.at[pl.ds(r0, RT)],
                            outd_hbm.at[c, pl.ds(r0, RT)])

    out, outd = body(node_embs, edge_embs, gcat, scat)
    return out[0], out[1], outd[0], outd[1]


def _tc_combine(acc_o, acc_i, deg_o, deg_i, node_embs,
                W_O_w, W_I_w, W_S_w, bias_sum, gamma, beta):
    def body(ao_ref, ai_ref, do_ref, di_ref, nd_ref, wo_ref, wi_ref, ws_ref,
             bs_ref, g_ref, b_ref, out_ref):
        deg_in = jnp.maximum(do_ref[:, 0:1], 1.0)
        deg_out = jnp.maximum(di_ref[:, 0:1], 1.0)
        ho = ao_ref[...] / deg_in
        hi = ai_ref[...] / deg_out
        h = (jnp.dot(ho, wo_ref[...].T, preferred_element_type=jnp.float32)
             + jnp.dot(hi, wi_ref[...].T, preferred_element_type=jnp.float32)
             + jnp.dot(nd_ref[...], ws_ref[...].T,
                       preferred_element_type=jnp.float32)
             + bs_ref[...]) / 3.0
        m = jnp.mean(h, axis=0, keepdims=True)
        hc = h - m
        v = jnp.mean(hc * hc, axis=0, keepdims=True)
        out_ref[...] = hc * lax.rsqrt(v + 1e-5) * g_ref[...] + b_ref[...]

    return pl.pallas_call(
        body,
        out_shape=jax.ShapeDtypeStruct((N, D), jnp.float32),
    )(acc_o, acc_i, deg_o, deg_i, node_embs, W_O_w, W_I_w, W_S_w,
      bias_sum, gamma, beta)


def kernel(node_embs, edge_embs, edge_index, W_O_w, W_O_b, W_I_w, W_I_b,
           W_S_w, W_S_b, gamma, beta):
    gcat = jnp.concatenate([edge_index[0], edge_index[1]])
    scat = jnp.concatenate([edge_index[1], edge_index[0]])
    acc_o, acc_i, deg_o, deg_i = _sc_aggregate(
        node_embs, edge_embs, gcat, scat)
    bias_sum = (W_O_b + W_I_b + W_S_b).reshape(1, D)
    return _tc_combine(acc_o, acc_i, deg_o, deg_i, node_embs,
                       W_O_w, W_I_w, W_S_w, bias_sum,
                       gamma.reshape(1, D), beta.reshape(1, D))


# speedup vs baseline: 1.1951x; 1.0004x over previous
import functools

import jax
import jax.numpy as jnp
from jax import lax
from jax.experimental import pallas as pl
from jax.experimental.pallas import tpu as pltpu
from jax.experimental.pallas import tpu_sc as plsc

N = 10000
E = 320000
D = 128

NC = 2
NS = 16
L = 16

EW = E // NS
C = 80
KCH = EW // C
RT = N // NS
ZB = 25


def _sc_aggregate(node_embs, edge_embs, gcat, scat):
    mesh = plsc.VectorSubcoreMesh(
        core_axis_name="c", subcore_axis_name="s",
        num_cores=NC, num_subcores=NS)

    out_type = [
        jax.ShapeDtypeStruct((NC, N, D), jnp.float32),
        jax.ShapeDtypeStruct((NC, N, L), jnp.float32),
    ]
    scratch = [
        pltpu.VMEM((2, C), jnp.int32),
        pltpu.VMEM((2, C), jnp.int32),
        pltpu.VMEM((2, C, D), jnp.float32),
        pltpu.VMEM((C, D), jnp.float32),
        pltpu.VMEM((C, L), jnp.float32),
        pltpu.VMEM((ZB, D), jnp.float32),
        pltpu.VMEM((ZB, L), jnp.float32),
        pltpu.VMEM_SHARED((N, D), jnp.float32),
        pltpu.VMEM_SHARED((N, L), jnp.float32),
        pltpu.SemaphoreType.DMA((2,)),
        pltpu.SemaphoreType.DMA((2,)),
        pltpu.SemaphoreType.DMA((2,)),
        pltpu.SemaphoreType.DMA((2,)),
        pltpu.SemaphoreType.DMA((2,)),
        pltpu.SemaphoreType.DMA,
    ]

    @functools.partial(
        pl.kernel, out_type=out_type, mesh=mesh, scratch_types=scratch,
        compiler_params=pltpu.CompilerParams(use_tc_tiling_on_sc=False))
    def body(node_hbm, edge_hbm, g_hbm, s_hbm,
             out_hbm, outd_hbm,
             gidx, sidx, gat, ech, ones, zbuf, zdbuf, acc, accd,
             gsem, igsem, issem, sasem, sdsem, esem):
        c = lax.axis_index("c")
        s = lax.axis_index("s")
        ebase = c * E + s * EW
        eoff = s * EW

        lanes = lax.iota(jnp.int32, L)
        zvec = jnp.zeros((L,), jnp.float32)
        onecol = jnp.where(lanes == 0, 1.0, 0.0)

        def initrow(r, _):
            ones[r, pl.ds(0, L)] = onecol
            return _
        lax.fori_loop(0, C, initrow, 0)

        def zrow(r, _):
            for cb in range(D // L):
                zbuf[r, pl.ds(cb * L, L)] = zvec
            zdbuf[r, pl.ds(0, L)] = zvec
            return _
        lax.fori_loop(0, ZB, zrow, 0)

        if True:
            with jax.named_scope("zero_acc"):
                for k in range(RT // ZB):
                    pltpu.sync_copy(zbuf, acc.at[pl.ds(s * RT + k * ZB, ZB)])
                    pltpu.sync_copy(zdbuf,
                                    accd.at[pl.ds(s * RT + k * ZB, ZB)])
                plsc.subcore_barrier()

            def issue_gidx(k, p):
                pltpu.async_copy(g_hbm.at[pl.ds(ebase + k * C, C)],
                                 gidx.at[p], igsem.at[p])

            def issue_sidx(k, p):
                pltpu.async_copy(s_hbm.at[pl.ds(ebase + k * C, C)],
                                 sidx.at[p], issem.at[p])

            def issue_gather(p):
                pltpu.async_copy(node_hbm.at[gidx.at[p]], gat.at[p],
                                 gsem.at[p])

            def issue_edge(k):
                pltpu.async_copy(edge_hbm.at[pl.ds(eoff + k * C, C)],
                                 ech, esem)

            def wait(src, dst, sem):
                pltpu.make_async_copy(src, dst, sem).wait()

            def subtract(p):
                @plsc.parallel_loop(0, C, 1, unroll=4)
                def _(r):
                    for cb in range(D // L):
                        sl = pl.ds(cb * L, L)
                        gat[p, r, sl] = gat[p, r, sl] - ech[r, sl]

            def issue_scatter(p):
                pltpu.async_copy(gat.at[p], acc.at[sidx.at[p]], sasem.at[p],
                                 add=True)
                pltpu.async_copy(ones, accd.at[sidx.at[p]], sdsem.at[p],
                                 add=True)

            def wait_scatter(p):
                wait(gat.at[p], acc.at[sidx.at[p]], sasem.at[p])
                wait(ones, accd.at[sidx.at[p]], sdsem.at[p])

            pltpu.sync_copy(g_hbm.at[pl.ds(ebase, C)], gidx.at[0])
            pltpu.sync_copy(s_hbm.at[pl.ds(ebase, C)], sidx.at[0])
            issue_gather(0)
            issue_edge(0)
            issue_gidx(1, 1)
            wait(node_hbm.at[gidx.at[0]], gat.at[0], gsem.at[0])
            wait(g_hbm.at[pl.ds(ebase, C)], gidx.at[1], igsem.at[1])
            issue_gather(1)
            issue_gidx(2, 0)
            issue_sidx(1, 1)
            wait(edge_hbm.at[pl.ds(eoff, C)], ech, esem)
            subtract(0)
            issue_edge(1)
            issue_scatter(0)

            def step(k, p, guard_i):
                q = 1 - p
                wait(node_hbm.at[gidx.at[p]], gat.at[p], gsem.at[p])
                wait_scatter(q)

                wait(g_hbm.at[pl.ds(ebase, C)], gidx.at[q], igsem.at[q])
                issue_gather(q)
                issue_sidx(k + 1, q)

                def prefetch_i():
                    issue_gidx(k + 2, p)
                if guard_i is None:
                    prefetch_i()
                else:
                    pl.when(guard_i)(prefetch_i)

                wait(edge_hbm.at[pl.ds(eoff, C)], ech, esem)
                subtract(p)
                issue_edge(k + 1)

                wait(s_hbm.at[pl.ds(ebase, C)], sidx.at[p], issem.at[p])
                issue_scatter(p)

            def pair(t, _):
                k1 = 2 * t + 1
                step(k1, 1, None)
                step(k1 + 1, 0, t < (KCH - 4) // 2)
                return _
            with jax.named_scope("main_loop"):
                lax.fori_loop(0, (KCH - 2) // 2, pair, 0)

            wait(node_hbm.at[gidx.at[1]], gat.at[1], gsem.at[1])
            wait_scatter(0)
            wait(edge_hbm.at[pl.ds(eoff, C)], ech, esem)
            subtract(1)
            wait(s_hbm.at[pl.ds(ebase, C)], sidx.at[1], issem.at[1])
            issue_scatter(1)
            wait_scatter(1)
            plsc.subcore_barrier()

            with jax.named_scope("copyout"):
                r0 = s * RT
                pltpu.sync_copy(acc.at[pl.ds(r0, RT)],
                                out_hbm.at[c, pl.ds(r0, RT)])
                pltpu.sync_copy(accd.at[pl.ds(r0, RT)],
                                outd_hbm.at[c, pl.ds(r0, RT)])

    out, outd = body(node_embs, edge_embs, gcat, scat)
    return out[0], out[1], outd[0], outd[1]


def _tc_combine(acc_o, acc_i, deg_o, deg_i, node_embs,
                W_O_w, W_I_w, W_S_w, bias_sum, gamma, beta):
    def body(ao_ref, ai_ref, do_ref, di_ref, nd_ref, wo_ref, wi_ref, ws_ref,
             bs_ref, g_ref, b_ref, out_ref):
        deg_in = jnp.maximum(do_ref[:, 0:1], 1.0)
        deg_out = jnp.maximum(di_ref[:, 0:1], 1.0)
        ho = ao_ref[...] / deg_in
        hi = ai_ref[...] / deg_out
        h = (jnp.dot(ho, wo_ref[...].T, preferred_element_type=jnp.float32)
             + jnp.dot(hi, wi_ref[...].T, preferred_element_type=jnp.float32)
             + jnp.dot(nd_ref[...], ws_ref[...].T,
                       preferred_element_type=jnp.float32)
             + bs_ref[...]) / 3.0
        m = jnp.mean(h, axis=0, keepdims=True)
        hc = h - m
        v = jnp.mean(hc * hc, axis=0, keepdims=True)
        out_ref[...] = hc * lax.rsqrt(v + 1e-5) * g_ref[...] + b_ref[...]

    return pl.pallas_call(
        body,
        out_shape=jax.ShapeDtypeStruct((N, D), jnp.float32),
    )(acc_o, acc_i, deg_o, deg_i, node_embs, W_O_w, W_I_w, W_S_w,
      bias_sum, gamma, beta)


def kernel(node_embs, edge_embs, edge_index, W_O_w, W_O_b, W_I_w, W_I_b,
           W_S_w, W_S_b, gamma, beta):
    gcat = jnp.concatenate([edge_index[0], edge_index[1]])
    scat = jnp.concatenate([edge_index[1], edge_index[0]])
    acc_o, acc_i, deg_o, deg_i = _sc_aggregate(
        node_embs, edge_embs, gcat, scat)
    bias_sum = (W_O_b + W_I_b + W_S_b).reshape(1, D)
    return _tc_combine(acc_o, acc_i, deg_o, deg_i, node_embs,
                       W_O_w, W_I_w, W_S_w, bias_sum,
                       gamma.reshape(1, D), beta.reshape(1, D))


# speedup vs baseline: 1.2269x; 1.0267x over previous
import functools

import jax
import jax.numpy as jnp
from jax import lax
from jax.experimental import pallas as pl
from jax.experimental.pallas import tpu as pltpu
from jax.experimental.pallas import tpu_sc as plsc

N = 10000
E = 320000
D = 128

NC = 2
NS = 16
L = 16

EW = E // NS
C = 80
KCH = EW // C
RT = N // NS
ZB = 25


def _sc_aggregate(node_embs, edge_embs, gcat, scat):
    mesh = plsc.VectorSubcoreMesh(
        core_axis_name="c", subcore_axis_name="s",
        num_cores=NC, num_subcores=NS)

    out_type = [
        jax.ShapeDtypeStruct((NC, N, D), jnp.float32),
        jax.ShapeDtypeStruct((NC, NS, N), jnp.float32),
    ]
    scratch = [
        pltpu.VMEM((2, C), jnp.int32),
        pltpu.VMEM((2, C), jnp.int32),
        pltpu.VMEM((2, C, D), jnp.float32),
        pltpu.VMEM((C, D), jnp.float32),
        pltpu.VMEM((ZB, D), jnp.float32),
        pltpu.VMEM((N,), jnp.float32),
        pltpu.VMEM_SHARED((N, D), jnp.float32),
        pltpu.SemaphoreType.DMA((2,)),
        pltpu.SemaphoreType.DMA((2,)),
        pltpu.SemaphoreType.DMA((2,)),
        pltpu.SemaphoreType.DMA((2,)),
        pltpu.SemaphoreType.DMA,
    ]

    @functools.partial(
        pl.kernel, out_type=out_type, mesh=mesh, scratch_types=scratch,
        compiler_params=pltpu.CompilerParams(use_tc_tiling_on_sc=False,
                                             needs_layout_passes=False))
    def body(node_hbm, edge_hbm, g_hbm, s_hbm,
             out_hbm, outd_hbm,
             gidx, sidx, gat, ech, zbuf, deg, acc,
             gsem, igsem, issem, sasem, esem):
        c = lax.axis_index("c")
        s = lax.axis_index("s")
        ebase = c * E + s * EW
        eoff = s * EW

        zvec = jnp.zeros((L,), jnp.float32)
        onev = zvec + 1.0

        def zrow(r, _):
            for cb in range(D // L):
                zbuf[r, pl.ds(cb * L, L)] = zvec
            return _
        lax.fori_loop(0, ZB, zrow, 0)

        def zdeg(i, _):
            deg[pl.ds(i * L, L)] = zvec
            return _
        lax.fori_loop(0, N // L, zdeg, 0)

        if True:
            with jax.named_scope("zero_acc"):
                for k in range(RT // ZB):
                    pltpu.sync_copy(zbuf, acc.at[pl.ds(s * RT + k * ZB, ZB)])
                plsc.subcore_barrier()

            def issue_gidx(k, p):
                pltpu.async_copy(g_hbm.at[pl.ds(ebase + k * C, C)],
                                 gidx.at[p], igsem.at[p])

            def issue_sidx(k, p):
                pltpu.async_copy(s_hbm.at[pl.ds(ebase + k * C, C)],
                                 sidx.at[p], issem.at[p])

            def issue_gather(p):
                pltpu.async_copy(node_hbm.at[gidx.at[p]], gat.at[p],
                                 gsem.at[p])

            def issue_edge(k):
                pltpu.async_copy(edge_hbm.at[pl.ds(eoff + k * C, C)],
                                 ech, esem)

            def wait(src, dst, sem):
                pltpu.make_async_copy(src, dst, sem).wait()

            def subtract(p):
                @plsc.parallel_loop(0, C, 1, unroll=4)
                def _(r):
                    for cb in range(D // L):
                        sl = pl.ds(cb * L, L)
                        gat[p, r, sl] = gat[p, r, sl] - ech[r, sl]

            def issue_scatter(p):
                pltpu.async_copy(gat.at[p], acc.at[sidx.at[p]], sasem.at[p],
                                 add=True)
                for v in range(C // L):
                    idxv = sidx[p, pl.ds(v * L, L)]
                    plsc.addupdate_scatter(deg, [idxv], onev)

            def wait_scatter(p):
                wait(gat.at[p], acc.at[sidx.at[p]], sasem.at[p])

            pltpu.sync_copy(g_hbm.at[pl.ds(ebase, C)], gidx.at[0])
            pltpu.sync_copy(s_hbm.at[pl.ds(ebase, C)], sidx.at[0])
            issue_gather(0)
            issue_edge(0)
            issue_gidx(1, 1)
            wait(node_hbm.at[gidx.at[0]], gat.at[0], gsem.at[0])
            wait(g_hbm.at[pl.ds(ebase, C)], gidx.at[1], igsem.at[1])
            issue_gather(1)
            issue_gidx(2, 0)
            issue_sidx(1, 1)
            wait(edge_hbm.at[pl.ds(eoff, C)], ech, esem)
            subtract(0)
            issue_edge(1)
            issue_scatter(0)

            def step(k, p, guard_i):
                q = 1 - p
                wait(node_hbm.at[gidx.at[p]], gat.at[p], gsem.at[p])
                wait_scatter(q)

                wait(g_hbm.at[pl.ds(ebase, C)], gidx.at[q], igsem.at[q])
                issue_gather(q)
                issue_sidx(k + 1, q)

                def prefetch_i():
                    issue_gidx(k + 2, p)
                if guard_i is None:
                    prefetch_i()
                else:
                    pl.when(guard_i)(prefetch_i)

                wait(edge_hbm.at[pl.ds(eoff, C)], ech, esem)
                subtract(p)
                issue_edge(k + 1)

                wait(s_hbm.at[pl.ds(ebase, C)], sidx.at[p], issem.at[p])
                issue_scatter(p)

            def pair(t, _):
                k1 = 2 * t + 1
                step(k1, 1, None)
                step(k1 + 1, 0, t < (KCH - 4) // 2)
                return _
            with jax.named_scope("main_loop"):
                lax.fori_loop(0, (KCH - 2) // 2, pair, 0)

            wait(node_hbm.at[gidx.at[1]], gat.at[1], gsem.at[1])
            wait_scatter(0)
            wait(edge_hbm.at[pl.ds(eoff, C)], ech, esem)
            subtract(1)
            wait(s_hbm.at[pl.ds(ebase, C)], sidx.at[1], issem.at[1])
            issue_scatter(1)
            wait_scatter(1)
            plsc.subcore_barrier()

            with jax.named_scope("copyout"):
                r0 = s * RT
                pltpu.sync_copy(acc.at[pl.ds(r0, RT)],
                                out_hbm.at[c, pl.ds(r0, RT)])
                pltpu.sync_copy(deg, outd_hbm.at[c, s])

    out, outd = body(node_embs, edge_embs, gcat, scat)
    return out[0], out[1], outd[0], outd[1]


def _tc_combine(acc_o, acc_i, deg_o, deg_i, node_embs,
                W_O_w, W_I_w, W_S_w, bias_sum, gamma, beta):
    def body(ao_ref, ai_ref, do_ref, di_ref, nd_ref, wo_ref, wi_ref, ws_ref,
             bs_ref, g_ref, b_ref, out_ref):
        deg_in = jnp.maximum(jnp.sum(do_ref[...], axis=0), 1.0).reshape(N, 1)
        deg_out = jnp.maximum(jnp.sum(di_ref[...], axis=0),
                              1.0).reshape(N, 1)
        ho = ao_ref[...] / deg_in
        hi = ai_ref[...] / deg_out
        h = (jnp.dot(ho, wo_ref[...].T, preferred_element_type=jnp.float32)
             + jnp.dot(hi, wi_ref[...].T, preferred_element_type=jnp.float32)
             + jnp.dot(nd_ref[...], ws_ref[...].T,
                       preferred_element_type=jnp.float32)
             + bs_ref[...]) / 3.0
        m = jnp.mean(h, axis=0, keepdims=True)
        hc = h - m
        v = jnp.mean(hc * hc, axis=0, keepdims=True)
        out_ref[...] = hc * lax.rsqrt(v + 1e-5) * g_ref[...] + b_ref[...]

    return pl.pallas_call(
        body,
        out_shape=jax.ShapeDtypeStruct((N, D), jnp.float32),
    )(acc_o, acc_i, deg_o, deg_i, node_embs, W_O_w, W_I_w, W_S_w,
      bias_sum, gamma, beta)


def kernel(node_embs, edge_embs, edge_index, W_O_w, W_O_b, W_I_w, W_I_b,
           W_S_w, W_S_b, gamma, beta):
    gcat = jnp.concatenate([edge_index[0], edge_index[1]])
    scat = jnp.concatenate([edge_index[1], edge_index[0]])
    acc_o, acc_i, deg_o, deg_i = _sc_aggregate(
        node_embs, edge_embs, gcat, scat)
    bias_sum = (W_O_b + W_I_b + W_S_b).reshape(1, D)
    return _tc_combine(acc_o, acc_i, deg_o, deg_i, node_embs,
                       W_O_w, W_I_w, W_S_w, bias_sum,
                       gamma.reshape(1, D), beta.reshape(1, D))


# speedup vs baseline: 1.2706x; 1.0356x over previous
import functools

import jax
import jax.numpy as jnp
from jax import lax
from jax.experimental import pallas as pl
from jax.experimental.pallas import tpu as pltpu
from jax.experimental.pallas import tpu_sc as plsc

N = 10000
E = 320000
D = 128

NC = 2
NS = 16
L = 16

EW = E // NS
C = 80
KCH = EW // C
RT = N // NS
ZB = 25


def _sc_aggregate(node_embs, edge_embs, eix_flat):
    mesh = plsc.VectorSubcoreMesh(
        core_axis_name="c", subcore_axis_name="s",
        num_cores=NC, num_subcores=NS)

    out_type = [
        jax.ShapeDtypeStruct((NC, N, D), jnp.float32),
        jax.ShapeDtypeStruct((NC, NS, N), jnp.float32),
    ]
    scratch = [
        pltpu.VMEM((2, C), jnp.int32),
        pltpu.VMEM((2, C), jnp.int32),
        pltpu.VMEM((2, C, D), jnp.float32),
        pltpu.VMEM((C, D), jnp.float32),
        pltpu.VMEM((ZB, D), jnp.float32),
        pltpu.VMEM((N,), jnp.float32),
        pltpu.VMEM_SHARED((N, D), jnp.float32),
        pltpu.SemaphoreType.DMA((2,)),
        pltpu.SemaphoreType.DMA((2,)),
        pltpu.SemaphoreType.DMA((2,)),
        pltpu.SemaphoreType.DMA((2,)),
        pltpu.SemaphoreType.DMA,
    ]

    @functools.partial(
        pl.kernel, out_type=out_type, mesh=mesh, scratch_types=scratch,
        compiler_params=pltpu.CompilerParams(use_tc_tiling_on_sc=False,
                                             needs_layout_passes=False))
    def body(node_hbm, edge_hbm, eix_hbm,
             out_hbm, outd_hbm,
             gidx, sidx, gat, ech, zbuf, deg, acc,
             gsem, igsem, issem, sasem, esem):
        c = lax.axis_index("c")
        s = lax.axis_index("s")
        eoff = s * EW
        gbase = c * E + eoff
        sbase = (1 - c) * E + eoff

        zvec = jnp.zeros((L,), jnp.float32)
        onev = zvec + 1.0

        def zrow(r, _):
            for cb in range(D // L):
                zbuf[r, pl.ds(cb * L, L)] = zvec
            return _
        lax.fori_loop(0, ZB, zrow, 0)

        def zdeg(i, _):
            deg[pl.ds(i * L, L)] = zvec
            return _
        lax.fori_loop(0, N // L, zdeg, 0)

        if True:
            with jax.named_scope("zero_acc"):
                for k in range(RT // ZB):
                    pltpu.sync_copy(zbuf, acc.at[pl.ds(s * RT + k * ZB, ZB)])
                plsc.subcore_barrier()

            def issue_gidx(k, p):
                pltpu.async_copy(eix_hbm.at[pl.ds(gbase + k * C, C)],
                                 gidx.at[p], igsem.at[p])

            def issue_sidx(k, p):
                pltpu.async_copy(eix_hbm.at[pl.ds(sbase + k * C, C)],
                                 sidx.at[p], issem.at[p])

            def issue_gather(p):
                pltpu.async_copy(node_hbm.at[gidx.at[p]], gat.at[p],
                                 gsem.at[p])

            def issue_edge(k):
                pltpu.async_copy(edge_hbm.at[pl.ds(eoff + k * C, C)],
                                 ech, esem)

            def wait(src, dst, sem):
                pltpu.make_async_copy(src, dst, sem).wait()

            def subtract(p):
                @plsc.parallel_loop(0, C, 1, unroll=4)
                def _(r):
                    for cb in range(D // L):
                        sl = pl.ds(cb * L, L)
                        gat[p, r, sl] = gat[p, r, sl] - ech[r, sl]

            def issue_scatter(p):
                pltpu.async_copy(gat.at[p], acc.at[sidx.at[p]], sasem.at[p],
                                 add=True)
                for v in range(C // L):
                    idxv = sidx[p, pl.ds(v * L, L)]
                    plsc.addupdate_scatter(deg, [idxv], onev)

            def wait_scatter(p):
                wait(gat.at[p], acc.at[sidx.at[p]], sasem.at[p])

            pltpu.sync_copy(eix_hbm.at[pl.ds(gbase, C)], gidx.at[0])
            pltpu.sync_copy(eix_hbm.at[pl.ds(sbase, C)], sidx.at[0])
            issue_gather(0)
            issue_edge(0)
            issue_gidx(1, 1)
            wait(node_hbm.at[gidx.at[0]], gat.at[0], gsem.at[0])
            wait(eix_hbm.at[pl.ds(gbase, C)], gidx.at[1], igsem.at[1])
            issue_gather(1)
            issue_gidx(2, 0)
            issue_sidx(1, 1)
            wait(edge_hbm.at[pl.ds(eoff, C)], ech, esem)
            subtract(0)
            issue_edge(1)
            issue_scatter(0)

            def step(k, p, guard_i):
                q = 1 - p
                wait(node_hbm.at[gidx.at[p]], gat.at[p], gsem.at[p])
                wait_scatter(q)

                wait(eix_hbm.at[pl.ds(gbase, C)], gidx.at[q], igsem.at[q])
                issue_gather(q)
                issue_sidx(k + 1, q)

                def prefetch_i():
                    issue_gidx(k + 2, p)
                if guard_i is None:
                    prefetch_i()
                else:
                    pl.when(guard_i)(prefetch_i)

                wait(edge_hbm.at[pl.ds(eoff, C)], ech, esem)
                subtract(p)
                issue_edge(k + 1)

                wait(eix_hbm.at[pl.ds(sbase, C)], sidx.at[p], issem.at[p])
                issue_scatter(p)

            def pair(t, _):
                k1 = 2 * t + 1
                step(k1, 1, None)
                step(k1 + 1, 0, t < (KCH - 4) // 2)
                return _
            with jax.named_scope("main_loop"):
                lax.fori_loop(0, (KCH - 2) // 2, pair, 0)

            wait(node_hbm.at[gidx.at[1]], gat.at[1], gsem.at[1])
            wait_scatter(0)
            wait(edge_hbm.at[pl.ds(eoff, C)], ech, esem)
            subtract(1)
            wait(eix_hbm.at[pl.ds(sbase, C)], sidx.at[1], issem.at[1])
            issue_scatter(1)
            wait_scatter(1)
            plsc.subcore_barrier()

            with jax.named_scope("copyout"):
                r0 = s * RT
                pltpu.sync_copy(acc.at[pl.ds(r0, RT)],
                                out_hbm.at[c, pl.ds(r0, RT)])
                pltpu.sync_copy(deg, outd_hbm.at[c, s])

    return body(node_embs, edge_embs, eix_flat)


def _tc_combine(accs, degs, node_embs,
                W_O_w, W_I_w, W_S_w, bias_sum, gamma, beta):
    def body(acc_ref, deg_ref, nd_ref, wo_ref, wi_ref, ws_ref,
             bs_ref, g_ref, b_ref, out_ref):
        deg_in = jnp.maximum(jnp.sum(deg_ref[0], axis=0), 1.0).reshape(N, 1)
        deg_out = jnp.maximum(jnp.sum(deg_ref[1], axis=0), 1.0).reshape(N, 1)
        ho = acc_ref[0] / deg_in
        hi = acc_ref[1] / deg_out
        h = (jnp.dot(ho, wo_ref[...].T, preferred_element_type=jnp.float32)
             + jnp.dot(hi, wi_ref[...].T, preferred_element_type=jnp.float32)
             + jnp.dot(nd_ref[...], ws_ref[...].T,
                       preferred_element_type=jnp.float32)
             + bs_ref[...]) / 3.0
        m = jnp.mean(h, axis=0, keepdims=True)
        hc = h - m
        v = jnp.mean(hc * hc, axis=0, keepdims=True)
        out_ref[...] = hc * lax.rsqrt(v + 1e-5) * g_ref[...] + b_ref[...]

    return pl.pallas_call(
        body,
        out_shape=jax.ShapeDtypeStruct((N, D), jnp.float32),
    )(accs, degs, node_embs, W_O_w, W_I_w, W_S_w, bias_sum, gamma, beta)


def kernel(node_embs, edge_embs, edge_index, W_O_w, W_O_b, W_I_w, W_I_b,
           W_S_w, W_S_b, gamma, beta):
    accs, degs = _sc_aggregate(node_embs, edge_embs,
                               edge_index.reshape(2 * E))
    bias_sum = (W_O_b + W_I_b + W_S_b).reshape(1, D)
    return _tc_combine(accs, degs, node_embs,
                       W_O_w, W_I_w, W_S_w, bias_sum,
                       gamma.reshape(1, D), beta.reshape(1, D))


# speedup vs baseline: 1.7431x; 1.3719x over previous
import functools

import jax
import jax.numpy as jnp
from jax import lax
from jax.experimental import pallas as pl
from jax.experimental.pallas import tpu as pltpu
from jax.experimental.pallas import tpu_sc as plsc

N = 10000
E = 320000
D = 128

NC = 2
NS = 16
L = 16

EW = E // NS
C = 80
KCH = EW // C
RT = N // NS
ZB = 25


def _sc_aggregate(node_b, edge_embs, eix_flat):
    mesh = plsc.VectorSubcoreMesh(
        core_axis_name="c", subcore_axis_name="s",
        num_cores=NC, num_subcores=NS)

    out_type = [
        jax.ShapeDtypeStruct((NC, N, D), jnp.float32),
        jax.ShapeDtypeStruct((NC, NS, N), jnp.float32),
    ]
    scratch = [
        pltpu.VMEM((2, C), jnp.int32),
        pltpu.VMEM((2, C), jnp.int32),
        pltpu.VMEM((2, C, D // 2), jnp.int32),
        pltpu.VMEM((2, C, D), jnp.float32),
        pltpu.VMEM((ZB, D), jnp.float32),
        pltpu.VMEM((N,), jnp.float32),
        pltpu.VMEM_SHARED((N, D), jnp.float32),
        pltpu.SemaphoreType.DMA((2,)),
        pltpu.SemaphoreType.DMA((2,)),
        pltpu.SemaphoreType.DMA((2,)),
        pltpu.SemaphoreType.DMA((2,)),
        pltpu.SemaphoreType.DMA((2,)),
    ]

    @functools.partial(
        pl.kernel, out_type=out_type, mesh=mesh, scratch_types=scratch,
        compiler_params=pltpu.CompilerParams(use_tc_tiling_on_sc=False,
                                             needs_layout_passes=False))
    def body(node_hbm, edge_hbm, eix_hbm,
             out_hbm, outd_hbm,
             gidx, sidx, gat, ech, zbuf, deg, acc,
             gsem, igsem, issem, sasem, esem):
        c = lax.axis_index("c")
        s = lax.axis_index("s")
        eoff = s * EW
        gbase = c * E + eoff
        sbase = (1 - c) * E + eoff

        zvec = jnp.zeros((L,), jnp.float32)
        onev = zvec + 1.0

        def zrow(r, _):
            for cb in range(D // L):
                zbuf[r, pl.ds(cb * L, L)] = zvec
            return _
        lax.fori_loop(0, ZB, zrow, 0)

        def zdeg(i, _):
            deg[pl.ds(i * L, L)] = zvec
            return _
        lax.fori_loop(0, N // L, zdeg, 0)

        if True:
            with jax.named_scope("zero_acc"):
                for k in range(RT // ZB):
                    pltpu.sync_copy(zbuf, acc.at[pl.ds(s * RT + k * ZB, ZB)])
                plsc.subcore_barrier()

            def issue_gidx(k, p):
                pltpu.async_copy(eix_hbm.at[pl.ds(gbase + k * C, C)],
                                 gidx.at[p], igsem.at[p])

            def issue_sidx(k, p):
                pltpu.async_copy(eix_hbm.at[pl.ds(sbase + k * C, C)],
                                 sidx.at[p], issem.at[p])

            def issue_gather(p):
                pltpu.async_copy(node_hbm.at[gidx.at[p]], gat.at[p],
                                 gsem.at[p])

            def issue_edge(k, p):
                pltpu.async_copy(edge_hbm.at[pl.ds(eoff + k * C, C)],
                                 ech.at[p], esem.at[p])

            def wait(src, dst, sem):
                pltpu.make_async_copy(src, dst, sem).wait()

            def subtract(p):
                @plsc.parallel_loop(0, C, 1, unroll=4)
                def _(r):
                    for cb in range(D // (2 * L)):
                        sl = pl.ds(cb * L, L)
                        sh = pl.ds(D // 2 + cb * L, L)
                        w = gat[p, r, sl]
                        lo = plsc.bitcast(w << 16, jnp.float32)
                        hi = plsc.bitcast(w & jnp.int32(-65536), jnp.float32)
                        ech[p, r, sl] = lo - ech[p, r, sl]
                        ech[p, r, sh] = hi - ech[p, r, sh]

            def issue_scatter(p):
                pltpu.async_copy(ech.at[p], acc.at[sidx.at[p]], sasem.at[p],
                                 add=True)
                for v in range(C // L):
                    idxv = sidx[p, pl.ds(v * L, L)]
                    plsc.addupdate_scatter(deg, [idxv], onev)

            def wait_scatter(p):
                wait(ech.at[p], acc.at[sidx.at[p]], sasem.at[p])

            pltpu.sync_copy(eix_hbm.at[pl.ds(gbase, C)], gidx.at[0])
            pltpu.sync_copy(eix_hbm.at[pl.ds(sbase, C)], sidx.at[0])
            issue_gather(0)
            issue_edge(0, 0)
            issue_gidx(1, 1)
            issue_sidx(1, 1)
            wait(node_hbm.at[gidx.at[0]], gat.at[0], gsem.at[0])
            wait(eix_hbm.at[pl.ds(gbase, C)], gidx.at[1], igsem.at[1])
            issue_gather(1)
            issue_gidx(2, 0)
            issue_edge(1, 1)
            wait(edge_hbm.at[pl.ds(eoff, C)], ech.at[0], esem.at[0])
            subtract(0)
            issue_scatter(0)

            def step(k, p, guard_i):
                q = 1 - p
                wait(node_hbm.at[gidx.at[p]], gat.at[p], gsem.at[p])
                wait(eix_hbm.at[pl.ds(gbase, C)], gidx.at[q], igsem.at[q])
                issue_gather(q)

                def prefetch_i():
                    issue_gidx(k + 2, p)
                if guard_i is None:
                    prefetch_i()
                else:
                    pl.when(guard_i)(prefetch_i)

                wait_scatter(q)
                issue_edge(k + 1, q)
                issue_sidx(k + 1, q)

                wait(edge_hbm.at[pl.ds(eoff, C)], ech.at[p], esem.at[p])
                subtract(p)

                wait(eix_hbm.at[pl.ds(sbase, C)], sidx.at[p], issem.at[p])
                issue_scatter(p)

            def pair(t, _):
                k1 = 2 * t + 1
                step(k1, 1, None)
                step(k1 + 1, 0, t < (KCH - 4) // 2)
                return _
            with jax.named_scope("main_loop"):
                lax.fori_loop(0, (KCH - 2) // 2, pair, 0)

            wait(node_hbm.at[gidx.at[1]], gat.at[1], gsem.at[1])
            wait_scatter(0)
            wait(edge_hbm.at[pl.ds(eoff, C)], ech.at[1], esem.at[1])
            subtract(1)
            wait(eix_hbm.at[pl.ds(sbase, C)], sidx.at[1], issem.at[1])
            issue_scatter(1)
            wait_scatter(1)
            plsc.subcore_barrier()

            with jax.named_scope("copyout"):
                r0 = s * RT
                pltpu.sync_copy(acc.at[pl.ds(r0, RT)],
                                out_hbm.at[c, pl.ds(r0, RT)])
                pltpu.sync_copy(deg, outd_hbm.at[c, s])

    return body(node_b, edge_embs, eix_flat)


def _tc_combine(accs, degs, node_embs,
                W_O_w, W_I_w, W_S_w, bias_sum, gamma, beta):
    def body(acc_ref, deg_ref, nd_ref, wo_ref, wi_ref, ws_ref,
             bs_ref, g_ref, b_ref, out_ref):
        deg_in = jnp.maximum(jnp.sum(deg_ref[0], axis=0), 1.0).reshape(N, 1)
        deg_out = jnp.maximum(jnp.sum(deg_ref[1], axis=0), 1.0).reshape(N, 1)
        ho = acc_ref[0] / deg_in
        hi = acc_ref[1] / deg_out
        h = (jnp.dot(ho, wo_ref[...].T, preferred_element_type=jnp.float32)
             + jnp.dot(hi, wi_ref[...].T, preferred_element_type=jnp.float32)
             + jnp.dot(nd_ref[...], ws_ref[...].T,
                       preferred_element_type=jnp.float32)
             + bs_ref[...]) / 3.0
        m = jnp.mean(h, axis=0, keepdims=True)
        hc = h - m
        v = jnp.mean(hc * hc, axis=0, keepdims=True)
        out_ref[...] = hc * lax.rsqrt(v + 1e-5) * g_ref[...] + b_ref[...]

    return pl.pallas_call(
        body,
        out_shape=jax.ShapeDtypeStruct((N, D), jnp.float32),
    )(accs, degs, node_embs, W_O_w, W_I_w, W_S_w, bias_sum, gamma, beta)


def kernel(node_embs, edge_embs, edge_index, W_O_w, W_O_b, W_I_w, W_I_b,
           W_S_w, W_S_b, gamma, beta):
    nb16 = node_embs.astype(jnp.bfloat16)
    node_b = jax.lax.bitcast_convert_type(
        jnp.stack([nb16[:, :D // 2], nb16[:, D // 2:]], axis=-1), jnp.int32)
    accs, degs = _sc_aggregate(node_b, edge_embs,
                               edge_index.reshape(2 * E))
    bias_sum = (W_O_b + W_I_b + W_S_b).reshape(1, D)
    return _tc_combine(accs, degs, node_embs,
                       W_O_w, W_I_w, W_S_w, bias_sum,
                       gamma.reshape(1, D), beta.reshape(1, D))
